# baseline (speedup 1.0000x reference)
import functools

import jax
import jax.numpy as jnp
from jax import lax
from jax.experimental import pallas as pl
from jax.experimental.pallas import tpu as pltpu
from jax.experimental.pallas import tpu_sc as plsc

N = 10000
C = 128
DEG = 32
NTILES = 16
NODES_T_FULL = 640
LAST_T_NODES = N - 15 * NODES_T_FULL
IDX_PER_T = NODES_T_FULL * DEG
STAGE_NODES = 40

_TC_BLK = 2000


def _tc_pre_body(x_ref, ip_ref, w_ref, u_ref):
    a = (ip_ref[:, 0:1] * w_ref[0:1, :]
         + ip_ref[:, 1:2] * w_ref[1:2, :]
         + ip_ref[:, 2:3] * w_ref[2:3, :])
    u_ref[...] = a * x_ref[...]


def _tc_post_body(su_ref, sx_ref, op_ref, w_ref, b_ref, o_ref):
    c = (op_ref[:, 0:1] * w_ref[3:4, :]
         + op_ref[:, 1:2] * w_ref[4:5, :]
         + op_ref[:, 2:3] * w_ref[5:6, :]
         + b_ref[...])
    o_ref[...] = (su_ref[0] + c * sx_ref[0]) * (1.0 / DEG)


def _sc_segsum_body(u_hbm, x_hbm, idx_hbm, out_hbm,
                    tbl_sh, idx_v, buf0, buf1, stage_v, sem0, sem1):
    cid = lax.axis_index("c")
    tid = lax.axis_index("s")

    @pl.when(tid == 0)
    def _():
        @pl.when(cid == 0)
        def _():
            pltpu.sync_copy(u_hbm, tbl_sh)

        @pl.when(cid == 1)
        def _():
            pltpu.sync_copy(x_hbm, tbl_sh)
    plsc.subcore_barrier()

    npt = jnp.where(tid == NTILES - 1, LAST_T_NODES, NODES_T_FULL)

    @pl.when(tid < NTILES - 1)
    def _():
        off = pl.multiple_of(tid * IDX_PER_T, 128)
        pltpu.sync_copy(idx_hbm.at[0, pl.ds(off, IDX_PER_T)], idx_v)

    @pl.when(tid == NTILES - 1)
    def _():
        pltpu.sync_copy(
            idx_hbm.at[0, pl.ds(15 * IDX_PER_T, LAST_T_NODES * DEG)],
            idx_v.at[pl.ds(0, LAST_T_NODES * DEG)])
    bufs = (buf0, buf1)
    sems = (sem0, sem1)

    def start(j, b):
        pltpu.async_copy(tbl_sh.at[idx_v.at[pl.ds(j * DEG, DEG)]],
                         bufs[b], sems[b])

    def wait(b):
        pltpu.make_async_copy(tbl_sh.at[idx_v.at[pl.ds(0, DEG)]],
                              bufs[b], sems[b]).wait()

    def process(j, b):
        wait(b)
        srow = j % STAGE_NODES
        for v in range(8):
            vals = [bufs[b][r, pl.ds(16 * v, 16)] for r in range(DEG)]
            while len(vals) > 1:
                vals = [vals[i] + vals[i + 1] for i in range(0, len(vals), 2)]
            stage_v[srow, pl.ds(16 * v, 16)] = vals[0]
        nxt = j + 2
        @pl.when(nxt < npt)
        def _():
            start(nxt, b)

        @pl.when(srow == STAGE_NODES - 1)
        def _():
            base = tid * NODES_T_FULL + (j // STAGE_NODES) * STAGE_NODES
            pltpu.sync_copy(stage_v, out_hbm.at[cid, pl.ds(base, STAGE_NODES)])

    start(0, 0)
    start(1, 1)

    def g_body(g, carry):
        process(2 * g, 0)
        process(2 * g + 1, 1)
        return carry
    lax.fori_loop(0, npt // 2, g_body, 0)


def kernel(x, in_points, out_points, neighbors_index, neighbors_row_splits, W, bias):
    del neighbors_row_splits
    x2 = x.reshape(N, C)
    ip2 = in_points.reshape(N, 3)
    op2 = out_points.reshape(N, 3)

    grid = N // _TC_BLK
    u_tbl = pl.pallas_call(
        _tc_pre_body,
        grid=(grid,),
        in_specs=[
            pl.BlockSpec((_TC_BLK, C), lambda i: (i, 0)),
            pl.BlockSpec((_TC_BLK, 3), lambda i: (i, 0)),
            pl.BlockSpec((6, C), lambda i: (0, 0)),
        ],
        out_specs=pl.BlockSpec((_TC_BLK, C), lambda i: (i, 0)),
        out_shape=jax.ShapeDtypeStruct((N, C), jnp.float32),
    )(x2, ip2, W)

    sc_segsum = pl.kernel(
        _sc_segsum_body,
        out_type=jax.ShapeDtypeStruct((2, N, C), jnp.float32),
        mesh=plsc.VectorSubcoreMesh(
            core_axis_name="c", subcore_axis_name="s",
            num_cores=2, num_subcores=NTILES),
        scratch_types=[
            pltpu.VMEM_SHARED((N, C), jnp.float32),
            pltpu.VMEM((IDX_PER_T,), jnp.int32),
            pltpu.VMEM((DEG, C), jnp.float32),
            pltpu.VMEM((DEG, C), jnp.float32),
            pltpu.VMEM((STAGE_NODES, C), jnp.float32),
            pltpu.SemaphoreType.DMA,
            pltpu.SemaphoreType.DMA,
        ],
    )
    sums = sc_segsum(u_tbl, x2, neighbors_index)

    out = pl.pallas_call(
        _tc_post_body,
        grid=(grid,),
        in_specs=[
            pl.BlockSpec((1, _TC_BLK, C), lambda i: (0, i, 0)),
            pl.BlockSpec((1, _TC_BLK, C), lambda i: (1, i, 0)),
            pl.BlockSpec((_TC_BLK, 3), lambda i: (i, 0)),
            pl.BlockSpec((6, C), lambda i: (0, 0)),
            pl.BlockSpec((1, C), lambda i: (0, 0)),
        ],
        out_specs=pl.BlockSpec((_TC_BLK, C), lambda i: (i, 0)),
        out_shape=jax.ShapeDtypeStruct((N, C), jnp.float32),
    )(sums, sums, op2, W, bias.reshape(1, C))

    return out.reshape(1, N, C)


# speedup vs baseline: 1.2642x; 1.2642x over previous
import functools

import jax
import jax.numpy as jnp
from jax import lax
from jax.experimental import pallas as pl
from jax.experimental.pallas import tpu as pltpu
from jax.experimental.pallas import tpu_sc as plsc

N = 10000
C = 128
DEG = 32
NTILES = 16
NODES_T_FULL = 640
LAST_T_NODES = N - 15 * NODES_T_FULL
IDX_PER_T = NODES_T_FULL * DEG
STAGE_NODES = 40

_TC_BLK = 2000


def _tc_pre_body(x_ref, ip_ref, w_ref, u_ref):
    a = (ip_ref[:, 0:1] * w_ref[0:1, :]
         + ip_ref[:, 1:2] * w_ref[1:2, :]
         + ip_ref[:, 2:3] * w_ref[2:3, :])
    u_ref[...] = a * x_ref[...]


def _tc_post_body(su_ref, sx_ref, op_ref, w_ref, b_ref, o_ref):
    c = (op_ref[:, 0:1] * w_ref[3:4, :]
         + op_ref[:, 1:2] * w_ref[4:5, :]
         + op_ref[:, 2:3] * w_ref[5:6, :]
         + b_ref[...])
    o_ref[...] = (su_ref[0] + c * sx_ref[0]) * (1.0 / DEG)


def _sc_segsum_body(u_hbm, x_hbm, idx_hbm, out_hbm,
                    tbl_sh, idx_v, buf0, buf1, stage_v, sem0, sem1):
    cid = lax.axis_index("c")
    tid = lax.axis_index("s")

    @pl.when(tid == 0)
    def _():
        @pl.when(cid == 0)
        def _():
            pltpu.sync_copy(u_hbm, tbl_sh)

        @pl.when(cid == 1)
        def _():
            pltpu.sync_copy(x_hbm, tbl_sh)
    plsc.subcore_barrier()

    npt = jnp.where(tid == NTILES - 1, LAST_T_NODES, NODES_T_FULL)

    @pl.when(tid < NTILES - 1)
    def _():
        off = pl.multiple_of(tid * IDX_PER_T, 128)
        pltpu.sync_copy(idx_hbm.at[0, pl.ds(off, IDX_PER_T)], idx_v)

    @pl.when(tid == NTILES - 1)
    def _():
        pltpu.sync_copy(
            idx_hbm.at[0, pl.ds(15 * IDX_PER_T, LAST_T_NODES * DEG)],
            idx_v.at[pl.ds(0, LAST_T_NODES * DEG)])
    bufs = (buf0, buf1)
    sems = (sem0, sem1)

    def start(j, b):
        pltpu.async_copy(tbl_sh.at[idx_v.at[pl.ds(j * DEG, DEG)]],
                         bufs[b], sems[b])

    def wait(b):
        pltpu.make_async_copy(tbl_sh.at[idx_v.at[pl.ds(0, DEG)]],
                              bufs[b], sems[b]).wait()

    def process(j, b):
        wait(b)
        srow = j % STAGE_NODES

        def row_loop(q, acc):
            r = q * 4
            out = []
            for v in range(8):
                s = pl.ds(16 * v, 16)
                t0 = bufs[b][r, s] + bufs[b][r + 1, s]
                t1 = bufs[b][r + 2, s] + bufs[b][r + 3, s]
                out.append(acc[v] + (t0 + t1))
            return tuple(out)
        acc0 = tuple(jnp.zeros((16,), jnp.float32) for _ in range(8))
        acc = lax.fori_loop(0, DEG // 4, row_loop, acc0)
        for v in range(8):
            stage_v[srow, pl.ds(16 * v, 16)] = acc[v]
        nxt = j + 2
        @pl.when(nxt < npt)
        def _():
            start(nxt, b)

        @pl.when(srow == STAGE_NODES - 1)
        def _():
            base = tid * NODES_T_FULL + (j // STAGE_NODES) * STAGE_NODES
            pltpu.sync_copy(stage_v, out_hbm.at[cid, pl.ds(base, STAGE_NODES)])

    start(0, 0)
    start(1, 1)

    def g_body(g, carry):
        process(2 * g, 0)
        process(2 * g + 1, 1)
        return carry
    lax.fori_loop(0, npt // 2, g_body, 0)


def kernel(x, in_points, out_points, neighbors_index, neighbors_row_splits, W, bias):
    del neighbors_row_splits
    x2 = x.reshape(N, C)
    ip2 = in_points.reshape(N, 3)
    op2 = out_points.reshape(N, 3)

    grid = N // _TC_BLK
    u_tbl = pl.pallas_call(
        _tc_pre_body,
        grid=(grid,),
        in_specs=[
            pl.BlockSpec((_TC_BLK, C), lambda i: (i, 0)),
            pl.BlockSpec((_TC_BLK, 3), lambda i: (i, 0)),
            pl.BlockSpec((6, C), lambda i: (0, 0)),
        ],
        out_specs=pl.BlockSpec((_TC_BLK, C), lambda i: (i, 0)),
        out_shape=jax.ShapeDtypeStruct((N, C), jnp.float32),
    )(x2, ip2, W)

    sc_segsum = pl.kernel(
        _sc_segsum_body,
        out_type=jax.ShapeDtypeStruct((2, N, C), jnp.float32),
        mesh=plsc.VectorSubcoreMesh(
            core_axis_name="c", subcore_axis_name="s",
            num_cores=2, num_subcores=NTILES),
        scratch_types=[
            pltpu.VMEM_SHARED((N, C), jnp.float32),
            pltpu.VMEM((IDX_PER_T,), jnp.int32),
            pltpu.VMEM((DEG, C), jnp.float32),
            pltpu.VMEM((DEG, C), jnp.float32),
            pltpu.VMEM((STAGE_NODES, C), jnp.float32),
            pltpu.SemaphoreType.DMA,
            pltpu.SemaphoreType.DMA,
        ],
    )
    sums = sc_segsum(u_tbl, x2, neighbors_index)

    out = pl.pallas_call(
        _tc_post_body,
        grid=(grid,),
        in_specs=[
            pl.BlockSpec((1, _TC_BLK, C), lambda i: (0, i, 0)),
            pl.BlockSpec((1, _TC_BLK, C), lambda i: (1, i, 0)),
            pl.BlockSpec((_TC_BLK, 3), lambda i: (i, 0)),
            pl.BlockSpec((6, C), lambda i: (0, 0)),
            pl.BlockSpec((1, C), lambda i: (0, 0)),
        ],
        out_specs=pl.BlockSpec((_TC_BLK, C), lambda i: (i, 0)),
        out_shape=jax.ShapeDtypeStruct((N, C), jnp.float32),
    )(sums, sums, op2, W, bias.reshape(1, C))

    return out.reshape(1, N, C)


# speedup vs baseline: 1.3790x; 1.0908x over previous
import functools

import jax
import jax.numpy as jnp
from jax import lax
from jax.experimental import pallas as pl
from jax.experimental.pallas import tpu as pltpu
from jax.experimental.pallas import tpu_sc as plsc

N = 10000
C = 128
DEG = 32
NTILES = 16
NODES_T_FULL = 640
LAST_T_NODES = N - 15 * NODES_T_FULL
IDX_PER_T = NODES_T_FULL * DEG
STAGE_NODES = 40

_TC_BLK = 2000


def _tc_pre_body(x_ref, ip_ref, w_ref, u_ref):
    a = (ip_ref[:, 0:1] * w_ref[0:1, :]
         + ip_ref[:, 1:2] * w_ref[1:2, :]
         + ip_ref[:, 2:3] * w_ref[2:3, :])
    u_ref[...] = a * x_ref[...]


def _tc_post_body(su_ref, sx_ref, op_ref, w_ref, b_ref, o_ref):
    c = (op_ref[:, 0:1] * w_ref[3:4, :]
         + op_ref[:, 1:2] * w_ref[4:5, :]
         + op_ref[:, 2:3] * w_ref[5:6, :]
         + b_ref[...])
    o_ref[...] = (su_ref[0] + c * sx_ref[0]) * (1.0 / DEG)


def _sc_segsum_body(u_hbm, x_hbm, idx_hbm, out_hbm,
                    tbl_sh, idx_v, buf0, buf1, stage_v, sem0, sem1):
    cid = lax.axis_index("c")
    tid = lax.axis_index("s")

    @pl.when(tid == 0)
    def _():
        @pl.when(cid == 0)
        def _():
            pltpu.sync_copy(u_hbm, tbl_sh)

        @pl.when(cid == 1)
        def _():
            pltpu.sync_copy(x_hbm, tbl_sh)
    plsc.subcore_barrier()

    npt = jnp.where(tid == NTILES - 1, LAST_T_NODES, NODES_T_FULL)

    @pl.when(tid < NTILES - 1)
    def _():
        off = pl.multiple_of(tid * IDX_PER_T, 128)
        pltpu.sync_copy(idx_hbm.at[0, pl.ds(off, IDX_PER_T)], idx_v)

    @pl.when(tid == NTILES - 1)
    def _():
        pltpu.sync_copy(
            idx_hbm.at[0, pl.ds(15 * IDX_PER_T, LAST_T_NODES * DEG)],
            idx_v.at[pl.ds(0, LAST_T_NODES * DEG)])
    bufs = (buf0, buf1)
    sems = (sem0, sem1)

    def start(j, b):
        pltpu.async_copy(tbl_sh.at[idx_v.at[pl.ds(j * DEG, DEG)]],
                         bufs[b], sems[b])

    def wait(b):
        pltpu.make_async_copy(tbl_sh.at[idx_v.at[pl.ds(0, DEG)]],
                              bufs[b], sems[b]).wait()

    def process(j, b):
        wait(b)
        srow = j % STAGE_NODES

        hi_mask = jnp.full((16,), -65536, dtype=jnp.int32)

        def row_loop(r, acc):
            out = []
            for v in range(4):
                w = bufs[b][r, pl.ds(16 * v, 16)]
                lo = jax.lax.bitcast_convert_type(w << 16, jnp.float32)
                hi = jax.lax.bitcast_convert_type(w & hi_mask, jnp.float32)
                out.append(acc[2 * v] + lo)
                out.append(acc[2 * v + 1] + hi)
            return tuple(out)
        acc0 = tuple(jnp.zeros((16,), jnp.float32) for _ in range(8))
        acc = lax.fori_loop(0, DEG, row_loop, acc0)
        for v in range(4):
            stage_v[srow, pl.ds(32 * v, 16)] = acc[2 * v]
            stage_v[srow, pl.ds(32 * v + 16, 16)] = acc[2 * v + 1]
        nxt = j + 2
        @pl.when(nxt < npt)
        def _():
            start(nxt, b)

        @pl.when(srow == STAGE_NODES - 1)
        def _():
            base = tid * NODES_T_FULL + (j // STAGE_NODES) * STAGE_NODES
            pltpu.sync_copy(stage_v, out_hbm.at[cid, pl.ds(base, STAGE_NODES)])

    start(0, 0)
    start(1, 1)

    def g_body(g, carry):
        process(2 * g, 0)
        process(2 * g + 1, 1)
        return carry
    lax.fori_loop(0, npt // 2, g_body, 0)


def kernel(x, in_points, out_points, neighbors_index, neighbors_row_splits, W, bias):
    del neighbors_row_splits
    x2 = x.reshape(N, C)
    ip2 = in_points.reshape(N, 3)
    op2 = out_points.reshape(N, 3)

    grid = N // _TC_BLK
    u_tbl = pl.pallas_call(
        _tc_pre_body,
        grid=(grid,),
        in_specs=[
            pl.BlockSpec((_TC_BLK, C), lambda i: (i, 0)),
            pl.BlockSpec((_TC_BLK, 3), lambda i: (i, 0)),
            pl.BlockSpec((6, C), lambda i: (0, 0)),
        ],
        out_specs=pl.BlockSpec((_TC_BLK, C), lambda i: (i, 0)),
        out_shape=jax.ShapeDtypeStruct((N, C), jnp.float32),
    )(x2, ip2, W)

    sc_segsum = pl.kernel(
        _sc_segsum_body,
        out_type=jax.ShapeDtypeStruct((2, N, C), jnp.float32),
        mesh=plsc.VectorSubcoreMesh(
            core_axis_name="c", subcore_axis_name="s",
            num_cores=2, num_subcores=NTILES),
        scratch_types=[
            pltpu.VMEM_SHARED((N, C // 2), jnp.int32),
            pltpu.VMEM((IDX_PER_T,), jnp.int32),
            pltpu.VMEM((DEG, C // 2), jnp.int32),
            pltpu.VMEM((DEG, C // 2), jnp.int32),
            pltpu.VMEM((STAGE_NODES, C), jnp.float32),
            pltpu.SemaphoreType.DMA,
            pltpu.SemaphoreType.DMA,
        ],
    )

    def _pack_bf16(t):
        t4 = t.astype(jnp.bfloat16).reshape(N, 4, 2, 16).swapaxes(2, 3)
        return jax.lax.bitcast_convert_type(t4, jnp.int32).reshape(N, C // 2)

    sums = sc_segsum(_pack_bf16(u_tbl), _pack_bf16(x2), neighbors_index)

    out = pl.pallas_call(
        _tc_post_body,
        grid=(grid,),
        in_specs=[
            pl.BlockSpec((1, _TC_BLK, C), lambda i: (0, i, 0)),
            pl.BlockSpec((1, _TC_BLK, C), lambda i: (1, i, 0)),
            pl.BlockSpec((_TC_BLK, 3), lambda i: (i, 0)),
            pl.BlockSpec((6, C), lambda i: (0, 0)),
            pl.BlockSpec((1, C), lambda i: (0, 0)),
        ],
        out_specs=pl.BlockSpec((_TC_BLK, C), lambda i: (i, 0)),
        out_shape=jax.ShapeDtypeStruct((N, C), jnp.float32),
    )(sums, sums, op2, W, bias.reshape(1, C))

    return out.reshape(1, N, C)


# speedup vs baseline: 1.3833x; 1.0031x over previous
import functools

import jax
import jax.numpy as jnp
from jax import lax
from jax.experimental import pallas as pl
from jax.experimental.pallas import tpu as pltpu
from jax.experimental.pallas import tpu_sc as plsc

N = 10000
C = 128
DEG = 32
NTILES = 16
NODES_T_FULL = 640
LAST_T_NODES = N - 15 * NODES_T_FULL
IDX_PER_T = NODES_T_FULL * DEG
STAGE_NODES = 40
NODES_PER_CHUNK = 2
CHUNKS_PER_STAGE = STAGE_NODES // NODES_PER_CHUNK

_TC_BLK = 2000


def _tc_pre_body(x_ref, ip_ref, w_ref, u_ref):
    a = (ip_ref[:, 0:1] * w_ref[0:1, :]
         + ip_ref[:, 1:2] * w_ref[1:2, :]
         + ip_ref[:, 2:3] * w_ref[2:3, :])
    u_ref[...] = a * x_ref[...]


def _tc_post_body(su_ref, sx_ref, op_ref, w_ref, b_ref, o_ref):
    c = (op_ref[:, 0:1] * w_ref[3:4, :]
         + op_ref[:, 1:2] * w_ref[4:5, :]
         + op_ref[:, 2:3] * w_ref[5:6, :]
         + b_ref[...])
    o_ref[...] = (su_ref[0] + c * sx_ref[0]) * (1.0 / DEG)


def _sc_segsum_body(u_hbm, x_hbm, idx_hbm, out_hbm,
                    tbl_sh, idx_v, buf0, buf1, stage_v, sem0, sem1):
    cid = lax.axis_index("c")
    tid = lax.axis_index("s")

    @pl.when(tid == 0)
    def _():
        @pl.when(cid == 0)
        def _():
            pltpu.sync_copy(u_hbm, tbl_sh)

        @pl.when(cid == 1)
        def _():
            pltpu.sync_copy(x_hbm, tbl_sh)
    plsc.subcore_barrier()

    npt = jnp.where(tid == NTILES - 1, LAST_T_NODES, NODES_T_FULL)

    @pl.when(tid < NTILES - 1)
    def _():
        off = pl.multiple_of(tid * IDX_PER_T, 128)
        pltpu.sync_copy(idx_hbm.at[0, pl.ds(off, IDX_PER_T)], idx_v)

    @pl.when(tid == NTILES - 1)
    def _():
        pltpu.sync_copy(
            idx_hbm.at[0, pl.ds(15 * IDX_PER_T, LAST_T_NODES * DEG)],
            idx_v.at[pl.ds(0, LAST_T_NODES * DEG)])
    bufs = (buf0, buf1)
    sems = (sem0, sem1)

    nchunks = npt // NODES_PER_CHUNK

    def start(j, b):
        pltpu.async_copy(
            tbl_sh.at[idx_v.at[pl.ds(j * NODES_PER_CHUNK * DEG,
                                     NODES_PER_CHUNK * DEG)]],
            bufs[b], sems[b])

    def wait(b):
        pltpu.make_async_copy(
            tbl_sh.at[idx_v.at[pl.ds(0, NODES_PER_CHUNK * DEG)]],
            bufs[b], sems[b]).wait()

    def process(j, b):
        wait(b)
        for t in range(NODES_PER_CHUNK):
            def row_loop(r, acc):
                row = t * DEG + r
                return tuple(acc[v] + bufs[b][row, pl.ds(16 * v, 16)]
                             for v in range(8))
            acc0 = tuple(jnp.zeros((16,), jnp.float32) for _ in range(8))
            acc = lax.fori_loop(0, DEG, row_loop, acc0)
            srow = (j % CHUNKS_PER_STAGE) * NODES_PER_CHUNK + t
            for v in range(8):
                stage_v[srow, pl.ds(16 * v, 16)] = acc[v]
        nxt = j + 2
        @pl.when(nxt < nchunks)
        def _():
            start(nxt, b)

        @pl.when(j % CHUNKS_PER_STAGE == CHUNKS_PER_STAGE - 1)
        def _():
            base = tid * NODES_T_FULL + (j // CHUNKS_PER_STAGE) * STAGE_NODES
            pltpu.sync_copy(stage_v, out_hbm.at[cid, pl.ds(base, STAGE_NODES)])

    start(0, 0)
    start(1, 1)

    def g_body(g, carry):
        process(2 * g, 0)
        process(2 * g + 1, 1)
        return carry
    lax.fori_loop(0, nchunks // 2, g_body, 0)


def kernel(x, in_points, out_points, neighbors_index, neighbors_row_splits, W, bias):
    del neighbors_row_splits
    x2 = x.reshape(N, C)
    ip2 = in_points.reshape(N, 3)
    op2 = out_points.reshape(N, 3)

    grid = N // _TC_BLK
    u_tbl = pl.pallas_call(
        _tc_pre_body,
        grid=(grid,),
        in_specs=[
            pl.BlockSpec((_TC_BLK, C), lambda i: (i, 0)),
            pl.BlockSpec((_TC_BLK, 3), lambda i: (i, 0)),
            pl.BlockSpec((6, C), lambda i: (0, 0)),
        ],
        out_specs=pl.BlockSpec((_TC_BLK, C), lambda i: (i, 0)),
        out_shape=jax.ShapeDtypeStruct((N, C), jnp.float32),
    )(x2, ip2, W)

    sc_segsum = pl.kernel(
        _sc_segsum_body,
        out_type=jax.ShapeDtypeStruct((2, N, C), jnp.float32),
        mesh=plsc.VectorSubcoreMesh(
            core_axis_name="c", subcore_axis_name="s",
            num_cores=2, num_subcores=NTILES),
        scratch_types=[
            pltpu.VMEM_SHARED((N, C), jnp.float32),
            pltpu.VMEM((IDX_PER_T,), jnp.int32),
            pltpu.VMEM((NODES_PER_CHUNK * DEG, C), jnp.float32),
            pltpu.VMEM((NODES_PER_CHUNK * DEG, C), jnp.float32),
            pltpu.VMEM((STAGE_NODES, C), jnp.float32),
            pltpu.SemaphoreType.DMA,
            pltpu.SemaphoreType.DMA,
        ],
    )

    sums = sc_segsum(u_tbl, x2, neighbors_index)

    out = pl.pallas_call(
        _tc_post_body,
        grid=(grid,),
        in_specs=[
            pl.BlockSpec((1, _TC_BLK, C), lambda i: (0, i, 0)),
            pl.BlockSpec((1, _TC_BLK, C), lambda i: (1, i, 0)),
            pl.BlockSpec((_TC_BLK, 3), lambda i: (i, 0)),
            pl.BlockSpec((6, C), lambda i: (0, 0)),
            pl.BlockSpec((1, C), lambda i: (0, 0)),
        ],
        out_specs=pl.BlockSpec((_TC_BLK, C), lambda i: (i, 0)),
        out_shape=jax.ShapeDtypeStruct((N, C), jnp.float32),
    )(sums, sums, op2, W, bias.reshape(1, C))

    return out.reshape(1, N, C)
